# Initial kernel scaffold; baseline (speedup 1.0000x reference)
import jax
import jax.numpy as jnp
from jax.experimental import pallas as pl


def kernel(features, edge_index, weight):
    raise NotImplementedError("write your pallas kernel here")


# speedup vs baseline: 8.4110x; 8.4110x over previous
import jax
import jax.numpy as jnp
from jax import lax
from jax.experimental import pallas as pl
from jax.experimental.pallas import tpu as pltpu
from jax.experimental.pallas import tpu_sc as plsc

N = 10000
E = 320000
D_IN = 128
D_OUT = 128

_NC = 2
_NS = 16
_NW = _NC * _NS
_CHUNK = 125
_NCHUNK = (E // _NW) // _CHUNK
_NPAD = 10240
_RPT = _NPAD // _NS
_ZR = 64

_MM_BLK = 1000


def _mm_body(f_ref, w_ref, o_ref):
    o_ref[...] = jnp.dot(f_ref[...], w_ref[...],
                         preferred_element_type=jnp.float32)


def _merge_body(p_ref, o_ref):
    o_ref[...] = jnp.maximum(p_ref[0] + p_ref[1], 0.0)


def _sc_body(support, cols3, rows3, out, cols_v, rows_v, gbuf, zbuf, acc, sem):
    c = lax.axis_index("c")
    s = lax.axis_index("s")
    wid = c * _NS + s

    def _z(t, carry):
        zbuf[t // 8, pl.ds((t % 8) * 16, 16)] = jnp.zeros((16,), jnp.float32)
        return carry
    lax.fori_loop(0, _ZR * 8, _z, 0)
    row0 = s * _RPT
    for k in range(_RPT // _ZR):
        pltpu.sync_copy(zbuf, acc.at[pl.ds(row0 + k * _ZR, _ZR)])
    plsc.subcore_barrier()

    pltpu.sync_copy(cols3.at[wid], cols_v)
    pltpu.sync_copy(rows3.at[wid], rows_v)

    def _edge(j, carry):
        pltpu.async_copy(support.at[cols_v.at[j]], gbuf, sem).wait()
        pltpu.sync_copy(gbuf, acc.at[rows_v.at[j]], add=True)
        return carry
    lax.fori_loop(0, _NCHUNK, _edge, 0)
    plsc.subcore_barrier()

    pltpu.sync_copy(acc.at[pl.ds(row0, _RPT)], out.at[c, pl.ds(row0, _RPT)])


def kernel(features, edge_index, weight):
    edge_index = edge_index.astype(jnp.int32)
    rows3 = edge_index[0].reshape(_NW, _NCHUNK, _CHUNK)
    cols3 = edge_index[1].reshape(_NW, _NCHUNK, _CHUNK)

    support = pl.pallas_call(
        _mm_body,
        grid=(N // _MM_BLK,),
        in_specs=[pl.BlockSpec((_MM_BLK, D_IN), lambda i: (i, 0)),
                  pl.BlockSpec((D_IN, D_OUT), lambda i: (0, 0))],
        out_specs=pl.BlockSpec((_MM_BLK, D_OUT), lambda i: (i, 0)),
        out_shape=jax.ShapeDtypeStruct((N, D_OUT), jnp.float32),
    )(features, weight)

    partials = pl.kernel(
        _sc_body,
        out_type=jax.ShapeDtypeStruct((_NC, _NPAD, D_OUT), jnp.float32),
        mesh=plsc.VectorSubcoreMesh(core_axis_name="c", subcore_axis_name="s"),
        scratch_types=[
            pltpu.VMEM((_NCHUNK, _CHUNK), jnp.int32),
            pltpu.VMEM((_NCHUNK, _CHUNK), jnp.int32),
            pltpu.VMEM((_CHUNK, D_OUT), jnp.float32),
            pltpu.VMEM((_ZR, D_OUT), jnp.float32),
            pltpu.VMEM_SHARED((_NPAD, D_OUT), jnp.float32),
            pltpu.SemaphoreType.DMA,
        ],
    )(support, cols3, rows3)

    return pl.pallas_call(
        _merge_body,
        grid=(N // _MM_BLK,),
        in_specs=[pl.BlockSpec((_NC, _MM_BLK, D_OUT), lambda i: (0, i, 0))],
        out_specs=pl.BlockSpec((_MM_BLK, D_OUT), lambda i: (i, 0)),
        out_shape=jax.ShapeDtypeStruct((N, D_OUT), jnp.float32),
    )(partials)
